# baseline (speedup 1.0000x reference)
import functools

import jax
import jax.numpy as jnp
from jax import lax
from jax.experimental import pallas as pl
from jax.experimental.pallas import tpu as pltpu
from jax.experimental.pallas import tpu_sc as plsc


def _sc_gather(emb, idx):
    B = idx.shape[0]
    D = emb.shape[1]
    info = plsc.get_sparse_core_info()
    nc, ns = info.num_cores, info.num_subcores
    nw = nc * ns
    b_per_w = B // nw
    mesh = plsc.VectorSubcoreMesh(core_axis_name="c", subcore_axis_name="s")

    @functools.partial(
        pl.kernel,
        mesh=mesh,
        out_type=jax.ShapeDtypeStruct((B, D), jnp.float32),
        scratch_types=[
            pltpu.VMEM((b_per_w,), jnp.int32),
            pltpu.VMEM((b_per_w, D), jnp.float32),
            pltpu.SemaphoreType.DMA,
        ],
        compiler_params=pltpu.CompilerParams(use_tc_tiling_on_sc=False),
    )
    def gather_kernel(table_hbm, idx_hbm, out_hbm, idx_v, rows_v, sem):
        wid = lax.axis_index("s") * nc + lax.axis_index("c")
        base = wid * b_per_w
        pltpu.sync_copy(idx_hbm.at[pl.ds(base, b_per_w)], idx_v)
        pltpu.async_copy(table_hbm.at[idx_v], rows_v, sem).wait()
        pltpu.sync_copy(rows_v, out_hbm.at[pl.ds(base, b_per_w)])

    return gather_kernel(emb, idx)


def _proj_kernel(x_ref, w_ref, b_ref, o_ref):
    o_ref[...] = lax.dot_general(
        x_ref[...], w_ref[...],
        (((1,), (1,)), ((), ())),
        preferred_element_type=jnp.float32,
    ) + b_ref[...]


def _tc_project(x, W, b2d, v_blk, interpret=False):
    n_tok, d = x.shape
    v = W.shape[0]
    grid = (pl.cdiv(v, v_blk),)
    return pl.pallas_call(
        _proj_kernel,
        grid=grid,
        in_specs=[
            pl.BlockSpec((n_tok, d), lambda i: (0, 0)),
            pl.BlockSpec((v_blk, d), lambda i: (i, 0)),
            pl.BlockSpec((1, v_blk), lambda i: (0, i)),
        ],
        out_specs=pl.BlockSpec((n_tok, v_blk), lambda i: (0, i)),
        out_shape=jax.ShapeDtypeStruct((n_tok, v), jnp.float32),
        compiler_params=pltpu.CompilerParams(
            dimension_semantics=("parallel",),
        ),
        interpret=interpret,
    )(x, W, b2d)


def kernel(tokens, emb, W, b):
    bsz, seq = tokens.shape
    v, d = emb.shape
    idx = tokens.reshape(bsz * seq).astype(jnp.int32)
    x = _sc_gather(emb, idx)
    y = _tc_project(x, W, b.reshape(1, v), v_blk=4096)
    return y.reshape(bsz, seq, v)


# speedup vs baseline: 1.2785x; 1.2785x over previous
import functools

import jax
import jax.numpy as jnp
from jax import lax
from jax.experimental import pallas as pl
from jax.experimental.pallas import tpu as pltpu
from jax.experimental.pallas import tpu_sc as plsc


def _sc_gather(emb, idx):
    B = idx.shape[0]
    D = emb.shape[1]
    info = plsc.get_sparse_core_info()
    nc, ns = info.num_cores, info.num_subcores
    nw = nc * ns
    b_per_w = B // nw
    mesh = plsc.VectorSubcoreMesh(core_axis_name="c", subcore_axis_name="s")

    @functools.partial(
        pl.kernel,
        mesh=mesh,
        out_type=jax.ShapeDtypeStruct((B, D), jnp.float32),
        scratch_types=[
            pltpu.VMEM((b_per_w,), jnp.int32),
            pltpu.VMEM((b_per_w, D), jnp.float32),
            pltpu.SemaphoreType.DMA,
        ],
        compiler_params=pltpu.CompilerParams(use_tc_tiling_on_sc=False),
    )
    def gather_kernel(table_hbm, idx_hbm, out_hbm, idx_v, rows_v, sem):
        wid = lax.axis_index("s") * nc + lax.axis_index("c")
        base = wid * b_per_w
        pltpu.sync_copy(idx_hbm.at[pl.ds(base, b_per_w)], idx_v)
        pltpu.async_copy(table_hbm.at[idx_v], rows_v, sem).wait()
        pltpu.sync_copy(rows_v, out_hbm.at[pl.ds(base, b_per_w)])

    return gather_kernel(emb, idx)


def _proj_kernel(x_ref, wt_ref, b_ref, o_ref):
    o_ref[...] = lax.dot_general(
        x_ref[...], wt_ref[...],
        (((1,), (0,)), ((), ())),
        preferred_element_type=jnp.float32,
    ) + b_ref[...]


def _tc_project(x, Wt, b2d, v_blk, interpret=False):
    n_tok, d = x.shape
    v = Wt.shape[1]
    grid = (pl.cdiv(v, v_blk),)
    return pl.pallas_call(
        _proj_kernel,
        grid=grid,
        in_specs=[
            pl.BlockSpec((n_tok, d), lambda i: (0, 0)),
            pl.BlockSpec((d, v_blk), lambda i: (0, i)),
            pl.BlockSpec((1, v_blk), lambda i: (0, i)),
        ],
        out_specs=pl.BlockSpec((n_tok, v_blk), lambda i: (0, i)),
        out_shape=jax.ShapeDtypeStruct((n_tok, v), jnp.float32),
        compiler_params=pltpu.CompilerParams(
            dimension_semantics=("parallel",),
        ),
        interpret=interpret,
    )(x, Wt, b2d)


def kernel(tokens, emb, W, b):
    bsz, seq = tokens.shape
    v, d = emb.shape
    idx = tokens.reshape(bsz * seq).astype(jnp.int32)
    x = _sc_gather(emb, idx)
    y = _tc_project(x, W.T, b.reshape(1, v), v_blk=4096)
    return y.reshape(bsz, seq, v)


# speedup vs baseline: 1.2830x; 1.0035x over previous
import functools

import jax
import jax.numpy as jnp
from jax import lax
from jax.experimental import pallas as pl
from jax.experimental.pallas import tpu as pltpu
from jax.experimental.pallas import tpu_sc as plsc


def _sc_gather_pairs(emb2, idx):
    B = idx.shape[0]
    D2 = emb2.shape[1]
    info = plsc.get_sparse_core_info()
    nc, ns = info.num_cores, info.num_subcores
    nw = nc * ns
    b_per_w = B // nw
    mesh = plsc.VectorSubcoreMesh(core_axis_name="c", subcore_axis_name="s")

    @functools.partial(
        pl.kernel,
        mesh=mesh,
        out_type=jax.ShapeDtypeStruct((B, D2), jnp.float32),
        scratch_types=[
            pltpu.VMEM((b_per_w,), jnp.int32),
            pltpu.VMEM((b_per_w, D2), jnp.float32),
            pltpu.SemaphoreType.DMA,
        ],
    )
    def gather_kernel(table_hbm, idx_hbm, out_hbm, idx_v, rows_v, sem):
        wid = lax.axis_index("s") * nc + lax.axis_index("c")
        base = wid * b_per_w
        pltpu.sync_copy(idx_hbm.at[pl.ds(base, b_per_w)], idx_v)
        ids = idx_v[...] >> 1
        pltpu.async_copy(table_hbm.at[ids], rows_v, sem).wait()
        pltpu.sync_copy(rows_v, out_hbm.at[pl.ds(base, b_per_w)])

    return gather_kernel(emb2, idx)


def _proj_kernel(tok_ref, x2_ref, wt_ref, b_ref, o_ref):
    d = wt_ref.shape[0]
    par = (tok_ref[...] & 1) == 1
    x = jnp.where(par, x2_ref[:, d:], x2_ref[:, :d])
    o_ref[...] = lax.dot_general(
        x, wt_ref[...],
        (((1,), (0,)), ((), ())),
        preferred_element_type=jnp.float32,
    ) + b_ref[...]


def _tc_project(idx2d, x2, Wt, b2d, v_blk):
    n_tok = x2.shape[0]
    d, v = Wt.shape
    grid = (pl.cdiv(v, v_blk),)
    return pl.pallas_call(
        _proj_kernel,
        grid=grid,
        in_specs=[
            pl.BlockSpec((n_tok, 1), lambda i: (0, 0)),
            pl.BlockSpec((n_tok, 2 * d), lambda i: (0, 0)),
            pl.BlockSpec((d, v_blk), lambda i: (0, i)),
            pl.BlockSpec((1, v_blk), lambda i: (0, i)),
        ],
        out_specs=pl.BlockSpec((n_tok, v_blk), lambda i: (0, i)),
        out_shape=jax.ShapeDtypeStruct((n_tok, v), jnp.float32),
        compiler_params=pltpu.CompilerParams(
            dimension_semantics=("parallel",),
        ),
    )(idx2d, x2, Wt, b2d)


def kernel(tokens, emb, W, b):
    bsz, seq = tokens.shape
    v, d = emb.shape
    idx = tokens.reshape(bsz * seq).astype(jnp.int32)
    emb2 = emb.reshape(v // 2, 2 * d)
    x2 = _sc_gather_pairs(emb2, idx)
    y = _tc_project(idx.reshape(-1, 1), x2, W.T, b.reshape(1, v), v_blk=4096)
    return y.reshape(bsz, seq, v)
